# baseline (speedup 1.0000x reference)
import jax
import jax.numpy as jnp
from jax.experimental import pallas as pl
from jax.experimental.pallas import tpu as pltpu

B, H, T_CACHE, D = 8, 16, 4096, 128
T_NEW = 2048
_NCHUNK = 8


def _body(ip_ref, pos_in_ref, kv_ref, vv_ref,
          k_out_ref, v_out_ref, pos_out_ref, sem_k, sem_v):
    for c in range(_NCHUNK):
        pltpu.make_async_copy(
            kv_ref.at[c], k_out_ref.at[c], sem_k.at[c]).start()
        pltpu.make_async_copy(
            vv_ref.at[c], v_out_ref.at[c], sem_v.at[c]).start()
    pos_out_ref[:, :T_NEW] = jnp.broadcast_to(ip_ref[...], (B, T_NEW))
    pos_out_ref[:, T_NEW:] = pos_in_ref[:, T_NEW:]
    for c in range(_NCHUNK):
        pltpu.make_async_copy(
            kv_ref.at[c], k_out_ref.at[c], sem_k.at[c]).wait()
        pltpu.make_async_copy(
            vv_ref.at[c], v_out_ref.at[c], sem_v.at[c]).wait()


def kernel(input_pos, k_val, v_val, k_cache, v_cache, pos):
    ip = input_pos.astype(jnp.int32).reshape(1, T_NEW)
    pos2d = pos.reshape(B, T_CACHE)
    kvc = k_val.reshape(_NCHUNK, (B * H) // _NCHUNK, T_NEW, D)
    vvc = v_val.reshape(_NCHUNK, (B * H) // _NCHUNK, T_NEW, D)

    k_out, v_out, pos_out = pl.pallas_call(
        _body,
        in_specs=[
            pl.BlockSpec(memory_space=pltpu.VMEM),
            pl.BlockSpec(memory_space=pltpu.VMEM),
            pl.BlockSpec(memory_space=pltpu.MemorySpace.HBM),
            pl.BlockSpec(memory_space=pltpu.MemorySpace.HBM),
        ],
        out_specs=[
            pl.BlockSpec(memory_space=pltpu.MemorySpace.HBM),
            pl.BlockSpec(memory_space=pltpu.MemorySpace.HBM),
            pl.BlockSpec(memory_space=pltpu.VMEM),
        ],
        out_shape=[
            jax.ShapeDtypeStruct(kvc.shape, k_val.dtype),
            jax.ShapeDtypeStruct(vvc.shape, v_val.dtype),
            jax.ShapeDtypeStruct((B, T_CACHE), jnp.int32),
        ],
        scratch_shapes=[
            pltpu.SemaphoreType.DMA((_NCHUNK,)),
            pltpu.SemaphoreType.DMA((_NCHUNK,)),
        ],
    )(ip, pos2d, kvc, vvc)

    k = k_out.reshape(B, H, T_NEW, D)
    v = v_out.reshape(B, H, T_NEW, D)
    return (k, v, pos_out.reshape(B, 1, T_CACHE))


# speedup vs baseline: 47.7517x; 47.7517x over previous
import jax
import jax.numpy as jnp
from jax.experimental import pallas as pl
from jax.experimental.pallas import tpu as pltpu

B, H, T_CACHE, D = 8, 16, 4096, 128
T_NEW = 2048
_ROWS = B * H * T_NEW
_BM = 8192
_GRID = _ROWS // _BM


def _body(ip_ref, pos_in_ref, kv_ref, vv_ref, k_out_ref, v_out_ref,
          pos_out_ref):
    k_out_ref[...] = kv_ref[...]
    v_out_ref[...] = vv_ref[...]

    @pl.when(pl.program_id(0) == 0)
    def _():
        pos_out_ref[:, :T_NEW] = jnp.broadcast_to(ip_ref[...], (B, T_NEW))
        pos_out_ref[:, T_NEW:] = pos_in_ref[:, T_NEW:]


def kernel(input_pos, k_val, v_val, k_cache, v_cache, pos):
    ip = input_pos.astype(jnp.int32).reshape(1, T_NEW)
    pos2d = pos.reshape(B, T_CACHE)
    kv2 = k_val.reshape(_ROWS, D)
    vv2 = v_val.reshape(_ROWS, D)

    k_out, v_out, pos_out = pl.pallas_call(
        _body,
        grid=(_GRID,),
        in_specs=[
            pl.BlockSpec((1, T_NEW), lambda i: (0, 0)),
            pl.BlockSpec((B, T_CACHE), lambda i: (0, 0)),
            pl.BlockSpec((_BM, D), lambda i: (i, 0)),
            pl.BlockSpec((_BM, D), lambda i: (i, 0)),
        ],
        out_specs=[
            pl.BlockSpec((_BM, D), lambda i: (i, 0)),
            pl.BlockSpec((_BM, D), lambda i: (i, 0)),
            pl.BlockSpec((B, T_CACHE), lambda i: (0, 0)),
        ],
        out_shape=[
            jax.ShapeDtypeStruct((_ROWS, D), k_val.dtype),
            jax.ShapeDtypeStruct((_ROWS, D), v_val.dtype),
            jax.ShapeDtypeStruct((B, T_CACHE), jnp.int32),
        ],
        compiler_params=pltpu.CompilerParams(
            dimension_semantics=("arbitrary",),
        ),
    )(ip, pos2d, kv2, vv2)

    k = k_out.reshape(B, H, T_NEW, D)
    v = v_out.reshape(B, H, T_NEW, D)
    return (k, v, pos_out.reshape(B, 1, T_CACHE))
